# baseline (speedup 1.0000x reference)
import functools

import jax
import jax.numpy as jnp
from jax import lax
from jax.experimental import pallas as pl
from jax.experimental.pallas import tpu as pltpu
from jax.experimental.pallas import tpu_sc as plsc

BATCH = 8
N_CLS = 50
N_CTX = 12
D = 768
CTX_LEN = 77
N_TAIL = 64
MIX = 16
HEAD = 1 + N_CTX
TILES_PER_B = 4
CLS_PER_TILE = 13
LANES = 16


def _vrow_copy(dst, drow, src, srow):
    for c in range(0, D, LANES):
        dst[drow, pl.ds(c, LANES)] = src[srow, pl.ds(c, LANES)]


def _body(ctx_hbm, pre_hbm, suf_hbm, emb_hbm, tok_hbm, nl_hbm, out_hbm,
          idx_v, ens_v, nl_v, pre_v, hd_v, suf_v,
          sem_g, sem_s, sem_m0, sem_m1, sem_h0, sem_h1, sem_w):
    nc = 2
    wid = lax.axis_index("s") * nc + lax.axis_index("c")
    b = wid // TILES_PER_B
    g = wid % TILES_PER_B
    i_start = g * CLS_PER_TILE
    i_count = jnp.minimum(CLS_PER_TILE, N_CLS - i_start)
    sem_m = (sem_m0, sem_m1)
    sem_h = (sem_h0, sem_h1)

    pltpu.sync_copy(tok_hbm.at[b], idx_v)
    pltpu.async_copy(emb_hbm.at[idx_v], ens_v, sem_g).wait()
    pltpu.sync_copy(ctx_hbm.at[b], hd_v.at[pl.ds(1, N_CTX)])
    pltpu.sync_copy(ctx_hbm.at[b], hd_v.at[pl.ds(HEAD + 1, N_CTX)])
    pltpu.sync_copy(pre_hbm.at[pl.ds(i_start, CLS_PER_TILE)], pre_v)
    pltpu.sync_copy(nl_hbm, nl_v)

    def suf_fetch(j, par):
        return pltpu.make_async_copy(
            suf_hbm.at[i_start + j, pl.ds(0, MIX)],
            suf_v.at[pl.ds(par * MIX, MIX)], sem_s)

    def mix_write(j, par):
        n = b * N_CLS + i_start + j
        return pltpu.make_async_copy(
            suf_v.at[pl.ds(par * MIX, MIX)],
            out_hbm.at[n, pl.ds(1 + N_CTX, MIX)], sem_m[par])

    def head_write(j, par):
        n = b * N_CLS + i_start + j
        return pltpu.make_async_copy(
            hd_v.at[pl.ds(par * HEAD, HEAD)],
            out_hbm.at[n, pl.ds(0, HEAD)], sem_h[par])

    suf_fetch(0, 0).start()

    def process(j, par):
        i = i_start + j
        n = b * N_CLS + i
        ell = nl_v[pl.ds(i, 16)][0]
        suf_fetch(j, par).wait()

        def overlay(p, carry):
            _vrow_copy(suf_v, par * MIX + p, ens_v, p - ell)
            return carry

        lax.fori_loop(ell, MIX, overlay, 0)

        @pl.when(j >= 2)
        def _():
            head_write(j, par).wait()

        _vrow_copy(hd_v, par * HEAD, pre_v, j)
        head_write(j, par).start()
        mix_write(j, par).start()
        pltpu.make_async_copy(
            ens_v.at[pl.ds(MIX - ell, N_TAIL - MIX)],
            out_hbm.at[n, pl.ds(HEAD + MIX, N_TAIL - MIX)], sem_w).start()

        @pl.when(j + 1 < i_count)
        def _():
            @pl.when(j >= 1)
            def _():
                mix_write(j - 1, 1 - par).wait()

            suf_fetch(j + 1, 1 - par).start()

    def outer(k, carry):
        j0 = 2 * k

        @pl.when(j0 < i_count)
        def _():
            process(j0, 0)

        @pl.when(j0 + 1 < i_count)
        def _():
            process(j0 + 1, 1)

        return carry

    lax.fori_loop(0, (CLS_PER_TILE + 1) // 2, outer, 0)

    mix_write(0, 0).wait()
    mix_write(0, 1).wait()
    head_write(0, 0).wait()
    head_write(0, 1).wait()

    def drain(j, carry):
        n = b * N_CLS + i_start + j
        pltpu.make_async_copy(
            ens_v.at[pl.ds(0, N_TAIL - MIX)],
            out_hbm.at[n, pl.ds(HEAD + MIX, N_TAIL - MIX)], sem_w).wait()
        return carry

    lax.fori_loop(0, i_count, drain, 0)


def kernel(ctx, token_prefix, token_suffix, embed_table, tokenized_ensemble,
           name_lens, tokenized_prompts):
    tok64 = tokenized_ensemble[:, :N_TAIL]
    pre2d = jnp.zeros((N_CLS + CLS_PER_TILE, D), jnp.float32)
    pre2d = pre2d.at[:N_CLS].set(token_prefix.reshape(N_CLS, D))
    nl64 = jnp.zeros((80,), jnp.int32).at[:N_CLS].set(name_lens)

    mesh = plsc.VectorSubcoreMesh(core_axis_name="c", subcore_axis_name="s")
    call = functools.partial(
        pl.kernel,
        mesh=mesh,
        compiler_params=pltpu.CompilerParams(use_tc_tiling_on_sc=False),
        out_type=jax.ShapeDtypeStruct((BATCH * N_CLS, CTX_LEN, D), jnp.float32),
        scratch_types=[
            pltpu.VMEM((N_TAIL,), jnp.int32),
            pltpu.VMEM((N_TAIL, D), jnp.float32),
            pltpu.VMEM((80,), jnp.int32),
            pltpu.VMEM((CLS_PER_TILE, D), jnp.float32),
            pltpu.VMEM((2 * HEAD, D), jnp.float32),
            pltpu.VMEM((2 * MIX, D), jnp.float32),
            pltpu.SemaphoreType.DMA,
            pltpu.SemaphoreType.DMA,
            pltpu.SemaphoreType.DMA,
            pltpu.SemaphoreType.DMA,
            pltpu.SemaphoreType.DMA,
            pltpu.SemaphoreType.DMA,
            pltpu.SemaphoreType.DMA,
        ],
    )(_body)
    prompts = call(ctx, pre2d, token_suffix, embed_table, tok64, nl64)

    tp = jnp.broadcast_to(tokenized_prompts[None],
                          (BATCH, N_CLS, CTX_LEN)).reshape(BATCH * N_CLS, CTX_LEN)
    return (prompts, tp)
